# Initial kernel scaffold; baseline (speedup 1.0000x reference)
import jax
import jax.numpy as jnp
from jax.experimental import pallas as pl


def kernel(x, pe_table):
    raise NotImplementedError("write your pallas kernel here")


# speedup vs baseline: 1.7196x; 1.7196x over previous
import jax
import jax.numpy as jnp
from jax.experimental import pallas as pl


_SB = 512


def _add_kernel(x_ref, pe_ref, o_ref):
    o_ref[...] = x_ref[...] + pe_ref[...][None, :, :]


def kernel(x, pe_table):
    batch, seq_len, d = x.shape
    nb = seq_len // _SB
    return pl.pallas_call(
        _add_kernel,
        grid=(nb,),
        in_specs=[
            pl.BlockSpec((batch, _SB, d), lambda i: (0, i, 0)),
            pl.BlockSpec((_SB, d), lambda i: (i, 0)),
        ],
        out_specs=pl.BlockSpec((batch, _SB, d), lambda i: (0, i, 0)),
        out_shape=jax.ShapeDtypeStruct((batch, seq_len, d), x.dtype),
    )(x, pe_table)
